# baseline (speedup 1.0000x reference)
import functools

import jax
import jax.numpy as jnp
from jax import lax
from jax.experimental import pallas as pl
from jax.experimental.pallas import tpu as pltpu
from jax.experimental.pallas import tpu_sc as plsc

B = 32
D_IN = 4096
D_HID = 32768
K = 64
BLK_E = 2048
BLK_D = 2048


def _encode_body(x_ref, w_ref, b_ref, r_ref):
    acc = jax.lax.dot_general(
        x_ref[...], w_ref[...], (((1,), (1,)), ((), ())),
        preferred_element_type=jnp.float32)
    pre = acc + b_ref[...].astype(jnp.float32)
    r_ref[...] = jnp.maximum(pre, 0.0)


def _key(r, col0, blk):
    bits = jax.lax.bitcast_convert_type(r, jnp.int32)
    col = col0 + jax.lax.broadcasted_iota(jnp.int32, (B, blk), 1)
    return (bits & jnp.int32(-65536)) | (D_HID - 1 - col)


def _hist_pick(hist_ref, target):
    lanes = lax.broadcasted_iota(jnp.int32, (16,), 0)
    bucket = jnp.int32(0)
    above = jnp.int32(0)
    cum = jnp.int32(0)
    for c in range(15, -1, -1):
        hv = hist_ref[pl.ds(c * 16, 16)]
        suf = lax.rev(jnp.cumsum(lax.rev(hv, (0,)), axis=0), (0,))
        s = suf + cum
        cross = (s >= target) & ((s - hv) < target)
        hit = jnp.max(cross.astype(jnp.int32)) > 0
        b_here = jnp.max(jnp.where(cross, c * 16 + lanes, 0))
        a_here = jnp.max(jnp.where(cross, s - hv, 0))
        bucket = jnp.where(hit, b_here, bucket)
        above = jnp.where(hit, a_here, above)
        cum = cum + jnp.sum(hv)
    return bucket, above


def _sc_thresh_body(r_hbm, tv_hbm, row_v, cand_v, hist_v, out_v):
    wid = lax.axis_index("s") * 2 + lax.axis_index("c")
    pltpu.sync_copy(r_hbm.at[wid], row_v)

    lanes = lax.broadcasted_iota(jnp.int32, (16,), 0)
    ones = jnp.ones((16,), jnp.int32)
    n_vreg = D_HID // 16
    unroll = 8
    n_pages = 4

    def _load_key(jj):
        v = row_v[pl.ds(jj * 16, 16)]
        bits = lax.bitcast_convert_type(v, jnp.int32)
        return (bits & jnp.int32(-65536)) | (D_HID - 1 - (jj * 16 + lanes))

    def _zero_hist(nwords):
        for c in range(nwords // 16):
            hist_v[pl.ds(c * 16, 16)] = jnp.zeros((16,), jnp.int32)

    neg_inf = jnp.full((16,), -2147483647 - 1, jnp.int32)

    def _insert(ts, key):
        t0, t1, t2, t3 = ts
        w = key
        hi = jnp.maximum(t0, w); w = jnp.minimum(t0, w); t0 = hi
        hi = jnp.maximum(t1, w); w = jnp.minimum(t1, w); t1 = hi
        hi = jnp.maximum(t2, w); w = jnp.minimum(t2, w); t2 = hi
        t3 = jnp.maximum(t3, w)
        return (t0, t1, t2, t3)

    def warm(j, ts):
        for u in range(unroll):
            ts = _insert(ts, _load_key(j * unroll + u))
        return ts

    ts = lax.fori_loop(0, 16, warm, (neg_inf, neg_inf, neg_inf, neg_inf))
    bound0 = jnp.min(ts[3])

    def comp0(j, pv):
        for u in range(unroll):
            key = _load_key(j * unroll + u)
            m = key >= bound0
            plsc.store_scatter(cand_v, [pv * 16 + lanes], key, mask=m)
            pv = pv + m.astype(jnp.int32)
        return pv

    pv = lax.fori_loop(0, 16, comp0, jnp.zeros((16,), jnp.int32))

    def main8(j, pv):
        for u in range(unroll):
            key = _load_key(j * unroll + u)
            m = key >= bound0
            plsc.store_scatter(cand_v, [pv * 16 + lanes], key, mask=m)
            pv = pv + m.astype(jnp.int32)
        return pv

    pvec = lax.fori_loop(16, n_vreg // unroll, main8, pv)
    ncand = jnp.max(pvec)

    def level(bshift, bmask, pshift, pmask, prefix, target):
        _zero_hist(512)

        def scan(j, _):
            key = cand_v[pl.ds(j * 16, 16)]
            valid = pvec > j
            if pshift is None:
                m = valid
            else:
                m = valid & ((lax.shift_right_logical(key, pshift) & pmask)
                             == prefix)
            b = lax.shift_right_logical(key, bshift) & bmask
            plsc.addupdate_scatter(hist_v, [b], ones, mask=m)
            return 0

        lax.fori_loop(0, ncand, scan, 0)
        return _hist_pick(hist_v, target)

    e1, gt1 = level(23, 511, None, None, None, jnp.int32(K))
    e2, gt2 = level(15, 255, 23, 511, e1, jnp.int32(K) - gt1)
    e3, gt3 = level(7, 255, 15, 0xFFFF, (e1 << 8) | e2,
                    jnp.int32(K) - gt1 - gt2)
    e4, _ = level(0, 127, 7, 0xFFFFFF, (e1 << 16) | (e2 << 8) | e3,
                  jnp.int32(K) - gt1 - gt2 - gt3)
    t = (e1 << 23) | (e2 << 15) | (e3 << 7) | e4

    tvec = jnp.full((16,), 0, jnp.int32) + t
    for c in range(8):
        out_v[pl.ds(c * 16, 16)] = tvec
    pltpu.sync_copy(out_v, tv_hbm.at[wid])


def _decode_body(r_ref, tv_ref, w_ref, b_ref, f_ref, o_ref, acc_ref):
    k = pl.program_id(0)
    r = r_ref[...]
    mask = _key(r, k * BLK_D, BLK_D) >= tv_ref[:, 0:1]
    feats = jnp.where(mask, r.astype(jnp.bfloat16), jnp.bfloat16(0))
    f_ref[...] = feats
    part = jax.lax.dot_general(
        feats, w_ref[...], (((1,), (1,)), ((), ())),
        preferred_element_type=jnp.float32)

    @pl.when(k == 0)
    def _():
        acc_ref[...] = part

    @pl.when(k > 0)
    def _():
        acc_ref[...] += part

    @pl.when(k == pl.num_programs(0) - 1)
    def _():
        o_ref[...] = (acc_ref[...] + b_ref[...].astype(jnp.float32)
                      ).astype(jnp.bfloat16)


def kernel(x, W_enc, b_enc, W_dec, b_dec):
    r = pl.pallas_call(
        _encode_body,
        grid=(D_HID // BLK_E,),
        in_specs=[
            pl.BlockSpec((B, D_IN), lambda k: (0, 0)),
            pl.BlockSpec((BLK_E, D_IN), lambda k: (k, 0)),
            pl.BlockSpec((1, BLK_E), lambda k: (0, k)),
        ],
        out_specs=pl.BlockSpec((B, BLK_E), lambda k: (0, k)),
        out_shape=jax.ShapeDtypeStruct((B, D_HID), jnp.float32),
    )(x, W_enc, b_enc.reshape(1, D_HID))

    tv = pl.kernel(
        _sc_thresh_body,
        mesh=plsc.VectorSubcoreMesh(core_axis_name="c", subcore_axis_name="s"),
        compiler_params=pltpu.CompilerParams(needs_layout_passes=False),
        out_type=jax.ShapeDtypeStruct((B, 128), jnp.int32),
        scratch_types=[
            pltpu.VMEM((D_HID,), jnp.float32),
            pltpu.VMEM((D_HID + 16,), jnp.int32),
            pltpu.VMEM((2048,), jnp.int32),
            pltpu.VMEM((128,), jnp.int32),
        ],
    )(r)

    f, o = pl.pallas_call(
        _decode_body,
        grid=(D_HID // BLK_D,),
        in_specs=[
            pl.BlockSpec((B, BLK_D), lambda k: (0, k)),
            pl.BlockSpec((B, 128), lambda k: (0, 0)),
            pl.BlockSpec((D_IN, BLK_D), lambda k: (0, k)),
            pl.BlockSpec((1, D_IN), lambda k: (0, 0)),
        ],
        out_specs=[
            pl.BlockSpec((B, BLK_D), lambda k: (0, k)),
            pl.BlockSpec((B, D_IN), lambda k: (0, 0)),
        ],
        out_shape=[
            jax.ShapeDtypeStruct((B, D_HID), jnp.bfloat16),
            jax.ShapeDtypeStruct((B, D_IN), jnp.bfloat16),
        ],
        scratch_shapes=[pltpu.VMEM((B, D_IN), jnp.float32)],
    )(r, tv, W_dec, b_dec.reshape(1, D_IN))

    return (o, f)


# speedup vs baseline: 1.0277x; 1.0277x over previous
import functools

import jax
import jax.numpy as jnp
from jax import lax
from jax.experimental import pallas as pl
from jax.experimental.pallas import tpu as pltpu
from jax.experimental.pallas import tpu_sc as plsc

B = 32
D_IN = 4096
D_HID = 32768
K = 64
BLK_E = 2048
BLK_D = 2048


def _encode_body(x_ref, w_ref, b_ref, r_ref):
    acc = jax.lax.dot_general(
        x_ref[...], w_ref[...], (((1,), (1,)), ((), ())),
        preferred_element_type=jnp.float32)
    pre = acc + b_ref[...].astype(jnp.float32)
    r_ref[...] = jnp.maximum(pre, 0.0)


def _key(r, col0, blk):
    bits = jax.lax.bitcast_convert_type(r, jnp.int32)
    col = col0 + jax.lax.broadcasted_iota(jnp.int32, (B, blk), 1)
    return (bits & jnp.int32(-65536)) | (D_HID - 1 - col)


def _hist_pick(hist_ref, target):
    lanes = lax.broadcasted_iota(jnp.int32, (16,), 0)
    bucket = jnp.int32(0)
    above = jnp.int32(0)
    cum = jnp.int32(0)
    for c in range(15, -1, -1):
        hv = hist_ref[pl.ds(c * 16, 16)]
        suf = lax.rev(jnp.cumsum(lax.rev(hv, (0,)), axis=0), (0,))
        s = suf + cum
        cross = (s >= target) & ((s - hv) < target)
        hit = jnp.max(cross.astype(jnp.int32)) > 0
        b_here = jnp.max(jnp.where(cross, c * 16 + lanes, 0))
        a_here = jnp.max(jnp.where(cross, s - hv, 0))
        bucket = jnp.where(hit, b_here, bucket)
        above = jnp.where(hit, a_here, above)
        cum = cum + jnp.sum(hv)
    return bucket, above


def _sc_thresh_body(r_hbm, tv_hbm, row_v, cand_v, hist_v, out_v):
    wid = lax.axis_index("s") * 2 + lax.axis_index("c")
    pltpu.sync_copy(r_hbm.at[wid], row_v)

    lanes = lax.broadcasted_iota(jnp.int32, (16,), 0)
    ones = jnp.ones((16,), jnp.int32)
    n_vreg = D_HID // 16
    unroll = 8
    n_pages = 4

    def _load_key(jj):
        v = row_v[pl.ds(jj * 16, 16)]
        bits = lax.bitcast_convert_type(v, jnp.int32)
        return (bits & jnp.int32(-65536)) | (D_HID - 1 - (jj * 16 + lanes))

    def _zero_hist(nwords):
        for c in range(nwords // 16):
            hist_v[pl.ds(c * 16, 16)] = jnp.zeros((16,), jnp.int32)

    neg_inf = jnp.full((16,), -2147483647 - 1, jnp.int32)

    def _insert(ts, key):
        t0, t1, t2, t3 = ts
        w = key
        hi = jnp.maximum(t0, w); w = jnp.minimum(t0, w); t0 = hi
        hi = jnp.maximum(t1, w); w = jnp.minimum(t1, w); t1 = hi
        hi = jnp.maximum(t2, w); w = jnp.minimum(t2, w); t2 = hi
        t3 = jnp.maximum(t3, w)
        return (t0, t1, t2, t3)

    def warm(j, ts):
        for u in range(unroll):
            ts = _insert(ts, _load_key(j * unroll + u))
        return ts

    ts = lax.fori_loop(0, 16, warm, (neg_inf, neg_inf, neg_inf, neg_inf))
    bound0 = jnp.min(ts[3])

    def comp0(j, pv):
        for u in range(unroll):
            key = _load_key(j * unroll + u)
            m = key >= bound0
            plsc.store_scatter(cand_v, [pv * 16 + lanes], key, mask=m)
            pv = pv + m.astype(jnp.int32)
        return pv

    pv = lax.fori_loop(0, 16, comp0, jnp.zeros((16,), jnp.int32))

    def sblock(s, carry):
        t0, t1, t2, t3, pv, bound = carry

        def m8(j, c):
            t0, t1, t2, t3, pv = c
            for u in range(unroll):
                key = _load_key((s * 16 + j) * unroll + u)
                t0, t1, t2, t3 = _insert((t0, t1, t2, t3), key)
                m = key >= bound
                plsc.store_scatter(cand_v, [pv * 16 + lanes], key, mask=m)
                pv = pv + m.astype(jnp.int32)
            return (t0, t1, t2, t3, pv)

        t0, t1, t2, t3, pv = lax.fori_loop(0, 16, m8, (t0, t1, t2, t3, pv))
        return (t0, t1, t2, t3, pv, jnp.min(t3))

    out = lax.fori_loop(1, n_vreg // unroll // 16, sblock, (*ts, pv, bound0))
    pvec = out[4]
    ncand = jnp.max(pvec)

    def level(bshift, bmask, pshift, pmask, prefix, target):
        _zero_hist(512)

        def scan(j, _):
            key = cand_v[pl.ds(j * 16, 16)]
            valid = pvec > j
            if pshift is None:
                m = valid
            else:
                m = valid & ((lax.shift_right_logical(key, pshift) & pmask)
                             == prefix)
            b = lax.shift_right_logical(key, bshift) & bmask
            plsc.addupdate_scatter(hist_v, [b], ones, mask=m)
            return 0

        lax.fori_loop(0, ncand, scan, 0)
        return _hist_pick(hist_v, target)

    e1, gt1 = level(23, 511, None, None, None, jnp.int32(K))
    e2, gt2 = level(15, 255, 23, 511, e1, jnp.int32(K) - gt1)
    e3, gt3 = level(7, 255, 15, 0xFFFF, (e1 << 8) | e2,
                    jnp.int32(K) - gt1 - gt2)
    e4, _ = level(0, 127, 7, 0xFFFFFF, (e1 << 16) | (e2 << 8) | e3,
                  jnp.int32(K) - gt1 - gt2 - gt3)
    t = (e1 << 23) | (e2 << 15) | (e3 << 7) | e4

    tvec = jnp.full((16,), 0, jnp.int32) + t
    for c in range(8):
        out_v[pl.ds(c * 16, 16)] = tvec
    pltpu.sync_copy(out_v, tv_hbm.at[wid])


def _decode_body(r_ref, tv_ref, w_ref, b_ref, f_ref, o_ref, acc_ref):
    k = pl.program_id(0)
    r = r_ref[...]
    mask = _key(r, k * BLK_D, BLK_D) >= tv_ref[:, 0:1]
    feats = jnp.where(mask, r.astype(jnp.bfloat16), jnp.bfloat16(0))
    f_ref[...] = feats
    part = jax.lax.dot_general(
        feats, w_ref[...], (((1,), (1,)), ((), ())),
        preferred_element_type=jnp.float32)

    @pl.when(k == 0)
    def _():
        acc_ref[...] = part

    @pl.when(k > 0)
    def _():
        acc_ref[...] += part

    @pl.when(k == pl.num_programs(0) - 1)
    def _():
        o_ref[...] = (acc_ref[...] + b_ref[...].astype(jnp.float32)
                      ).astype(jnp.bfloat16)


def kernel(x, W_enc, b_enc, W_dec, b_dec):
    r = pl.pallas_call(
        _encode_body,
        grid=(D_HID // BLK_E,),
        in_specs=[
            pl.BlockSpec((B, D_IN), lambda k: (0, 0)),
            pl.BlockSpec((BLK_E, D_IN), lambda k: (k, 0)),
            pl.BlockSpec((1, BLK_E), lambda k: (0, k)),
        ],
        out_specs=pl.BlockSpec((B, BLK_E), lambda k: (0, k)),
        out_shape=jax.ShapeDtypeStruct((B, D_HID), jnp.float32),
    )(x, W_enc, b_enc.reshape(1, D_HID))

    tv = pl.kernel(
        _sc_thresh_body,
        mesh=plsc.VectorSubcoreMesh(core_axis_name="c", subcore_axis_name="s"),
        compiler_params=pltpu.CompilerParams(needs_layout_passes=False),
        out_type=jax.ShapeDtypeStruct((B, 128), jnp.int32),
        scratch_types=[
            pltpu.VMEM((D_HID,), jnp.float32),
            pltpu.VMEM((D_HID + 16,), jnp.int32),
            pltpu.VMEM((2048,), jnp.int32),
            pltpu.VMEM((128,), jnp.int32),
        ],
    )(r)

    f, o = pl.pallas_call(
        _decode_body,
        grid=(D_HID // BLK_D,),
        in_specs=[
            pl.BlockSpec((B, BLK_D), lambda k: (0, k)),
            pl.BlockSpec((B, 128), lambda k: (0, 0)),
            pl.BlockSpec((D_IN, BLK_D), lambda k: (0, k)),
            pl.BlockSpec((1, D_IN), lambda k: (0, 0)),
        ],
        out_specs=[
            pl.BlockSpec((B, BLK_D), lambda k: (0, k)),
            pl.BlockSpec((B, D_IN), lambda k: (0, 0)),
        ],
        out_shape=[
            jax.ShapeDtypeStruct((B, D_HID), jnp.bfloat16),
            jax.ShapeDtypeStruct((B, D_IN), jnp.bfloat16),
        ],
        scratch_shapes=[pltpu.VMEM((B, D_IN), jnp.float32)],
    )(r, tv, W_dec, b_dec.reshape(1, D_IN))

    return (o, f)
